# baseline (speedup 1.0000x reference)
import functools
import math

import jax
import jax.numpy as jnp
from jax import lax
from jax.experimental import pallas as pl
from jax.experimental.pallas import tpu as pltpu
from jax.experimental.pallas import tpu_sc as plsc

_D = 64
_SCALE = math.sqrt(_D)
_B = 4096 * 200
_NC = 2
_NS = 16
_NW = _NC * _NS
_BPW = _B // _NW
_C = 512
_STEPS = _BPW // _C

_mesh = plsc.VectorSubcoreMesh(core_axis_name="c", subcore_axis_name="s")


@functools.partial(
    pl.kernel,
    out_type=jax.ShapeDtypeStruct((_B, _D), jnp.float32),
    mesh=_mesh,
    scratch_types=[
        pltpu.VMEM((_C,), jnp.int32),
        pltpu.VMEM((_C, _D), jnp.float32),
        pltpu.SemaphoreType.DMA,
    ],
    compiler_params=pltpu.CompilerParams(use_tc_tiling_on_sc=False),
)
def _emb_lookup(idx_hbm, table_hbm, out_hbm, idx_v, rows_v, gsem):
    wid = lax.axis_index("s") * _NC + lax.axis_index("c")
    base = wid * _BPW

    def step(s, carry):
        off = base + s * _C
        pltpu.sync_copy(idx_hbm.at[pl.ds(off, _C)], idx_v)
        pltpu.async_copy(table_hbm.at[idx_v], rows_v, gsem).wait()

        def srow(r, c):
            for d4 in range(_D // 16):
                sl = pl.ds(d4 * 16, 16)
                rows_v[r, sl] = rows_v[r, sl] * _SCALE
            return c

        lax.fori_loop(0, _C, srow, 0)
        pltpu.sync_copy(rows_v, out_hbm.at[pl.ds(off, _C)])
        return carry

    lax.fori_loop(0, _STEPS, step, 0)


def kernel(x, table):
    idx = x.reshape(-1).astype(jnp.int32)
    out = _emb_lookup(idx, table)
    return out.reshape(x.shape + (_D,))
